# Initial kernel scaffold; baseline (speedup 1.0000x reference)
import jax
import jax.numpy as jnp
from jax.experimental import pallas as pl


def kernel(features, labels, emb_table, bias_table, bias):
    raise NotImplementedError("write your pallas kernel here")


# speedup vs baseline: 1.3344x; 1.3344x over previous
import functools

import jax
import jax.numpy as jnp
from jax import lax
from jax.experimental import pallas as pl
from jax.experimental.pallas import tpu as pltpu
from jax.experimental.pallas import tpu_sc as plsc

F = 26
K = 16
C = 32
CF = C * F


def _permute(v, idx):
    dnums = lax.GatherDimensionNumbers(
        offset_dims=(), collapsed_slice_dims=(0,), start_index_map=(0,))
    return lax.gather(v, idx[:, None], dnums, slice_sizes=(1,),
                      mode=lax.GatherScatterMode.PROMISE_IN_BOUNDS)


def _fm_kernel(feat_hbm, emb_hbm, bias_hbm, out_hbm,
               idx_v, rows_v, bvals_v, out_v, sem_e, sem_b,
               *, nw, nc, rows_per_w, n_chunks):
    wid = lax.axis_index("s") * nc + lax.axis_index("c")

    bvals_v[pl.ds(CF, 16)] = jnp.zeros((16,), jnp.float32)

    def chunk_body(c, carry):
        lane = jnp.arange(16, dtype=jnp.int32)
        bmask = jnp.where(lane < (F - 16), jnp.float32(1.0), jnp.float32(0.0))

        def lane_sum(v):
            for shift in (8, 4, 2, 1):
                v = v + _permute(v, lane ^ shift)
            return v

        base_row = wid * rows_per_w + c * C
        fbase = base_row * F
        pltpu.sync_copy(feat_hbm.at[pl.ds(fbase, CF)], idx_v)
        cp_e = pltpu.async_copy(emb_hbm.at[idx_v], rows_v, sem_e)
        cp_b = pltpu.async_copy(bias_hbm.at[idx_v], bvals_v.at[pl.ds(0, CF)], sem_b)
        cp_e.wait()
        cp_b.wait()

        for g in range(C // 16):
            acc = jnp.zeros((16,), jnp.float32)
            for j in range(16):
                r = g * 16 + j
                e = rows_v[r * F]
                s = e
                sq = e * e
                for f in range(1, F):
                    e = rows_v[r * F + f]
                    s = s + e
                    sq = sq + e * e
                v = 0.5 * (s * s - sq)
                b1 = bvals_v[pl.ds(r * F, 16)]
                b2 = bvals_v[pl.ds(r * F + 16, 16)] * bmask
                tot = v + b1 + b2
                acc = jnp.where(lane == j, lane_sum(tot), acc)
            out_v[pl.ds(g * 16, 16)] = acc

        pltpu.sync_copy(out_v, out_hbm.at[pl.ds(base_row, C)])
        return carry

    lax.fori_loop(0, n_chunks, chunk_body, 0)


def kernel(features, labels, emb_table, bias_table, bias):
    B = features.shape[0]
    info = plsc.get_sparse_core_info()
    nc, ns = info.num_cores, info.num_subcores
    nw = nc * ns
    rows_per_w = B // nw
    n_chunks = rows_per_w // C

    feat_flat = features.reshape(-1).astype(jnp.int32)
    bias_flat = bias_table.reshape(-1)

    mesh = plsc.VectorSubcoreMesh(core_axis_name="c", subcore_axis_name="s")
    fm = pl.kernel(
        functools.partial(_fm_kernel, nw=nw, nc=nc,
                          rows_per_w=rows_per_w, n_chunks=n_chunks),
        mesh=mesh,
        compiler_params=pltpu.CompilerParams(use_tc_tiling_on_sc=False),
        out_type=jax.ShapeDtypeStruct((B,), jnp.float32),
        scratch_types=[
            pltpu.VMEM((CF,), jnp.int32),
            pltpu.VMEM((CF, K), jnp.float32),
            pltpu.VMEM((CF + 16,), jnp.float32),
            pltpu.VMEM((C,), jnp.float32),
            pltpu.SemaphoreType.DMA,
            pltpu.SemaphoreType.DMA,
        ],
    )
    out = fm(feat_flat, emb_table, bias_flat)
    return out.reshape(B, 1) + bias


# speedup vs baseline: 1.6608x; 1.2446x over previous
import functools

import jax
import jax.numpy as jnp
from jax import lax
from jax.experimental import pallas as pl
from jax.experimental.pallas import tpu as pltpu
from jax.experimental.pallas import tpu_sc as plsc

F = 26
K = 16
C = 32
CF = C * F


TR_BW = 8192


def _tr_kernel(in_ref, out_ref, scr):
    scr[...] = in_ref[...].T
    out_ref[...] = jnp.concatenate(
        [scr[pl.ds(r, TR_BW // 8, 8), :] for r in range(8)], axis=1)


def _row_major_table(emb_table):
    M = emb_table.shape[0]
    emb_t = jnp.swapaxes(emb_table, 0, 1)
    packed = pl.pallas_call(
        _tr_kernel,
        grid=(pl.cdiv(M, TR_BW),),
        in_specs=[pl.BlockSpec((16, TR_BW), lambda j: (0, j))],
        out_specs=pl.BlockSpec((TR_BW // 8, 128), lambda j: (j, 0)),
        out_shape=jax.ShapeDtypeStruct((M // 8, 128), jnp.float32),
        scratch_shapes=[pltpu.VMEM((TR_BW, 16), jnp.float32)],
    )(emb_t)
    return packed.reshape(M, 16)


def _permute(v, idx):
    dnums = lax.GatherDimensionNumbers(
        offset_dims=(), collapsed_slice_dims=(0,), start_index_map=(0,))
    return lax.gather(v, idx[:, None], dnums, slice_sizes=(1,),
                      mode=lax.GatherScatterMode.PROMISE_IN_BOUNDS)


def _fm_kernel(feat_hbm, emb_hbm, bias_hbm, out_hbm,
               idx_v, rows_v, bvals_v, out_v, sem_e, sem_b,
               *, nw, nc, rows_per_w, n_chunks):
    wid = lax.axis_index("s") * nc + lax.axis_index("c")

    bvals_v[pl.ds(CF, 16)] = jnp.zeros((16,), jnp.float32)

    def chunk_body(c, carry):
        lane = jnp.arange(16, dtype=jnp.int32)
        bmask = jnp.where(lane < (F - 16), jnp.float32(1.0), jnp.float32(0.0))

        def lane_sum(v):
            for shift in (8, 4, 2, 1):
                v = v + _permute(v, lane ^ shift)
            return v

        base_row = wid * rows_per_w + c * C
        fbase = base_row * F
        pltpu.sync_copy(feat_hbm.at[pl.ds(fbase, CF)], idx_v)
        cp_e = pltpu.async_copy(emb_hbm.at[idx_v], rows_v, sem_e)
        cp_b = pltpu.async_copy(bias_hbm.at[idx_v], bvals_v.at[pl.ds(0, CF)], sem_b)
        cp_e.wait()
        cp_b.wait()

        for g in range(C // 16):
            acc = jnp.zeros((16,), jnp.float32)
            for j in range(16):
                r = g * 16 + j
                e = rows_v[r * F]
                s = e
                sq = e * e
                for f in range(1, F):
                    e = rows_v[r * F + f]
                    s = s + e
                    sq = sq + e * e
                v = 0.5 * (s * s - sq)
                b1 = bvals_v[pl.ds(r * F, 16)]
                b2 = bvals_v[pl.ds(r * F + 16, 16)] * bmask
                tot = v + b1 + b2
                acc = jnp.where(lane == j, lane_sum(tot), acc)
            out_v[pl.ds(g * 16, 16)] = acc

        pltpu.sync_copy(out_v, out_hbm.at[pl.ds(base_row, C)])
        return carry

    lax.fori_loop(0, n_chunks, chunk_body, 0)


def kernel(features, labels, emb_table, bias_table, bias):
    B = features.shape[0]
    info = plsc.get_sparse_core_info()
    nc, ns = info.num_cores, info.num_subcores
    nw = nc * ns
    rows_per_w = B // nw
    n_chunks = rows_per_w // C

    feat_flat = features.reshape(-1).astype(jnp.int32)
    bias_flat = bias_table.reshape(-1)

    mesh = plsc.VectorSubcoreMesh(core_axis_name="c", subcore_axis_name="s")
    fm = pl.kernel(
        functools.partial(_fm_kernel, nw=nw, nc=nc,
                          rows_per_w=rows_per_w, n_chunks=n_chunks),
        mesh=mesh,
        compiler_params=pltpu.CompilerParams(use_tc_tiling_on_sc=False),
        out_type=jax.ShapeDtypeStruct((B,), jnp.float32),
        scratch_types=[
            pltpu.VMEM((CF,), jnp.int32),
            pltpu.VMEM((CF, K), jnp.float32),
            pltpu.VMEM((CF + 16,), jnp.float32),
            pltpu.VMEM((C,), jnp.float32),
            pltpu.SemaphoreType.DMA,
            pltpu.SemaphoreType.DMA,
        ],
    )
    out = fm(feat_flat, _row_major_table(emb_table), bias_flat)
    return out.reshape(B, 1) + bias


# speedup vs baseline: 1.6761x; 1.0092x over previous
import functools

import jax
import jax.numpy as jnp
from jax import lax
from jax.experimental import pallas as pl
from jax.experimental.pallas import tpu as pltpu
from jax.experimental.pallas import tpu_sc as plsc

F = 26
K = 16
C = 32
CF = C * F


TR_BW = 8192


def _tr_kernel(in_ref, out_ref, scr):
    scr[...] = in_ref[...].T
    out_ref[...] = jnp.concatenate(
        [scr[pl.ds(r, TR_BW // 8, 8), :] for r in range(8)], axis=1)


def _row_major_table(emb_table):
    M = emb_table.shape[0]
    emb_t = jnp.swapaxes(emb_table, 0, 1)
    packed = pl.pallas_call(
        _tr_kernel,
        grid=(pl.cdiv(M, TR_BW),),
        in_specs=[pl.BlockSpec((16, TR_BW), lambda j: (0, j))],
        out_specs=pl.BlockSpec((TR_BW // 8, 128), lambda j: (j, 0)),
        out_shape=jax.ShapeDtypeStruct((M // 8, 128), jnp.float32),
        scratch_shapes=[pltpu.VMEM((TR_BW, 16), jnp.float32)],
    )(emb_t)
    return packed.reshape(M, 16)


def _permute(v, idx):
    dnums = lax.GatherDimensionNumbers(
        offset_dims=(), collapsed_slice_dims=(0,), start_index_map=(0,))
    return lax.gather(v, idx[:, None], dnums, slice_sizes=(1,),
                      mode=lax.GatherScatterMode.PROMISE_IN_BOUNDS)


def _fm_kernel(feat_hbm, emb_hbm, bias_hbm, out_hbm,
               idx0, idx1, rows0, rows1, bv0, bv1, out_v,
               se0, sb0, se1, sb1, si0, si1,
               *, nw, nc, rows_per_w, n_chunks):
    wid = lax.axis_index("s") * nc + lax.axis_index("c")
    w_base = wid * rows_per_w

    bv0[pl.ds(CF, 16)] = jnp.zeros((16,), jnp.float32)
    bv1[pl.ds(CF, 16)] = jnp.zeros((16,), jnp.float32)

    def idx_cp(c, iv, sem):
        fbase = (w_base + c * C) * F
        return pltpu.make_async_copy(feat_hbm.at[pl.ds(fbase, CF)], iv, sem)

    def e_cp(iv, rv, sem):
        return pltpu.make_async_copy(emb_hbm.at[iv], rv, sem)

    def b_cp(iv, bv, sem):
        return pltpu.make_async_copy(bias_hbm.at[iv], bv.at[pl.ds(0, CF)], sem)

    def compute_chunk(c, rows_v, bvals_v):
        lane = jnp.arange(16, dtype=jnp.int32)
        bmask = jnp.where(lane < (F - 16), jnp.float32(1.0), jnp.float32(0.0))

        def lane_sum(v):
            for shift in (8, 4, 2, 1):
                v = v + _permute(v, lane ^ shift)
            return v

        for g in range(C // 16):
            acc = jnp.zeros((16,), jnp.float32)
            for j in range(16):
                r = g * 16 + j
                e = rows_v[r * F]
                s = e
                sq = e * e
                for f in range(1, F):
                    e = rows_v[r * F + f]
                    s = s + e
                    sq = sq + e * e
                v = 0.5 * (s * s - sq)
                b1 = bvals_v[pl.ds(r * F, 16)]
                b2 = bvals_v[pl.ds(r * F + 16, 16)] * bmask
                tot = v + b1 + b2
                acc = jnp.where(lane == j, lane_sum(tot), acc)
            out_v[pl.ds(g * 16, 16)] = acc

        pltpu.sync_copy(out_v, out_hbm.at[pl.ds(w_base + c * C, C)])

    def wrap(c):
        return jnp.where(c >= n_chunks, c - n_chunks, c)

    cp = idx_cp(0, idx0, si0)
    cp.start()
    cp.wait()
    e_cp(idx0, rows0, se0).start()
    b_cp(idx0, bv0, sb0).start()
    idx_cp(1, idx1, si1).start()

    def body(i, carry):
        c0 = 2 * i
        idx_cp(0, idx1, si1).wait()
        e_cp(idx1, rows1, se1).start()
        b_cp(idx1, bv1, sb1).start()

        e_cp(idx0, rows0, se0).wait()
        b_cp(idx0, bv0, sb0).wait()
        compute_chunk(c0, rows0, bv0)

        idx_cp(wrap(c0 + 2), idx0, si0).start()

        e_cp(idx1, rows1, se1).wait()
        b_cp(idx1, bv1, sb1).wait()
        compute_chunk(c0 + 1, rows1, bv1)

        idx_cp(0, idx0, si0).wait()
        e_cp(idx0, rows0, se0).start()
        b_cp(idx0, bv0, sb0).start()
        idx_cp(wrap(c0 + 3), idx1, si1).start()
        return carry

    lax.fori_loop(0, n_chunks // 2, body, 0)

    e_cp(idx0, rows0, se0).wait()
    b_cp(idx0, bv0, sb0).wait()
    idx_cp(0, idx1, si1).wait()


def kernel(features, labels, emb_table, bias_table, bias):
    B = features.shape[0]
    info = plsc.get_sparse_core_info()
    nc, ns = info.num_cores, info.num_subcores
    nw = nc * ns
    rows_per_w = B // nw
    n_chunks = rows_per_w // C

    feat_flat = features.reshape(-1).astype(jnp.int32)
    bias_flat = bias_table.reshape(-1)

    mesh = plsc.VectorSubcoreMesh(core_axis_name="c", subcore_axis_name="s")
    fm = pl.kernel(
        functools.partial(_fm_kernel, nw=nw, nc=nc,
                          rows_per_w=rows_per_w, n_chunks=n_chunks),
        mesh=mesh,
        compiler_params=pltpu.CompilerParams(use_tc_tiling_on_sc=False),
        out_type=jax.ShapeDtypeStruct((B,), jnp.float32),
        scratch_types=[
            pltpu.VMEM((CF,), jnp.int32),
            pltpu.VMEM((CF,), jnp.int32),
            pltpu.VMEM((CF, K), jnp.float32),
            pltpu.VMEM((CF, K), jnp.float32),
            pltpu.VMEM((CF + 16,), jnp.float32),
            pltpu.VMEM((CF + 16,), jnp.float32),
            pltpu.VMEM((C,), jnp.float32),
            pltpu.SemaphoreType.DMA,
            pltpu.SemaphoreType.DMA,
            pltpu.SemaphoreType.DMA,
            pltpu.SemaphoreType.DMA,
            pltpu.SemaphoreType.DMA,
            pltpu.SemaphoreType.DMA,
        ],
    )
    out = fm(feat_flat, _row_major_table(emb_table), bias_flat)
    return out.reshape(B, 1) + bias


# speedup vs baseline: 1.7043x; 1.0168x over previous
import functools

import jax
import jax.numpy as jnp
from jax import lax
from jax.experimental import pallas as pl
from jax.experimental.pallas import tpu as pltpu
from jax.experimental.pallas import tpu_sc as plsc

F = 26
K = 16
C = 16
CF = C * F


TR_BW = 8192


def _tr_kernel(in_ref, out_ref, scr):
    scr[...] = in_ref[...].T
    out_ref[...] = jnp.concatenate(
        [scr[pl.ds(r, TR_BW // 8, 8), :] for r in range(8)], axis=1)


def _row_major_table(emb_table):
    M = emb_table.shape[0]
    emb_t = jnp.swapaxes(emb_table, 0, 1)
    packed = pl.pallas_call(
        _tr_kernel,
        grid=(pl.cdiv(M, TR_BW),),
        in_specs=[pl.BlockSpec((16, TR_BW), lambda j: (0, j))],
        out_specs=pl.BlockSpec((TR_BW // 8, 128), lambda j: (j, 0)),
        out_shape=jax.ShapeDtypeStruct((M // 8, 128), jnp.float32),
        scratch_shapes=[pltpu.VMEM((TR_BW, 16), jnp.float32)],
    )(emb_t)
    return packed.reshape(M, 16)


def _permute(v, idx):
    dnums = lax.GatherDimensionNumbers(
        offset_dims=(), collapsed_slice_dims=(0,), start_index_map=(0,))
    return lax.gather(v, idx[:, None], dnums, slice_sizes=(1,),
                      mode=lax.GatherScatterMode.PROMISE_IN_BOUNDS)


def _fm_kernel(feat_hbm, emb_hbm, bias_hbm, out_hbm,
               idx0, idx1, rows0, rows1, bv0, bv1, out_v,
               se0, sb0, se1, sb1, si0, si1,
               *, nw, nc, rows_per_w, n_chunks):
    wid = lax.axis_index("s") * nc + lax.axis_index("c")
    w_base = wid * rows_per_w

    bv0[pl.ds(CF, 16)] = jnp.zeros((16,), jnp.float32)
    bv1[pl.ds(CF, 16)] = jnp.zeros((16,), jnp.float32)

    def idx_cp(c, iv, sem):
        fbase = (w_base + c * C) * F
        return pltpu.make_async_copy(feat_hbm.at[pl.ds(fbase, CF)], iv, sem)

    def e_cp(iv, rv, sem):
        return pltpu.make_async_copy(emb_hbm.at[iv], rv, sem)

    def b_cp(iv, bv, sem):
        return pltpu.make_async_copy(bias_hbm.at[iv], bv.at[pl.ds(0, CF)], sem)

    def compute_chunk(c, rows_v, bvals_v):
        lane = jnp.arange(16, dtype=jnp.int32)
        bmask = jnp.where(lane < (F - 16), jnp.float32(1.0), jnp.float32(0.0))

        def lane_sum(v):
            for shift in (8, 4, 2, 1):
                v = v + _permute(v, lane ^ shift)
            return v

        for g in range(C // 16):
            acc = jnp.zeros((16,), jnp.float32)
            for j in range(16):
                r = g * 16 + j
                e = rows_v[r * F]
                s = e
                sq = e * e
                for f in range(1, F):
                    e = rows_v[r * F + f]
                    s = s + e
                    sq = sq + e * e
                v = 0.5 * (s * s - sq)
                b1 = bvals_v[pl.ds(r * F, 16)]
                b2 = bvals_v[pl.ds(r * F + 16, 16)] * bmask
                tot = v + b1 + b2
                acc = jnp.where(lane == j, lane_sum(tot), acc)
            out_v[pl.ds(g * 16, 16)] = acc

        pltpu.sync_copy(out_v, out_hbm.at[pl.ds(w_base + c * C, C)])

    def wrap(c):
        return jnp.where(c >= n_chunks, c - n_chunks, c)

    cp = idx_cp(0, idx0, si0)
    cp.start()
    cp.wait()
    e_cp(idx0, rows0, se0).start()
    b_cp(idx0, bv0, sb0).start()
    idx_cp(1, idx1, si1).start()

    def body(i, carry):
        c0 = 2 * i
        idx_cp(0, idx1, si1).wait()
        e_cp(idx1, rows1, se1).start()
        b_cp(idx1, bv1, sb1).start()

        e_cp(idx0, rows0, se0).wait()
        b_cp(idx0, bv0, sb0).wait()
        compute_chunk(c0, rows0, bv0)

        idx_cp(wrap(c0 + 2), idx0, si0).start()

        e_cp(idx1, rows1, se1).wait()
        b_cp(idx1, bv1, sb1).wait()
        compute_chunk(c0 + 1, rows1, bv1)

        idx_cp(0, idx0, si0).wait()
        e_cp(idx0, rows0, se0).start()
        b_cp(idx0, bv0, sb0).start()
        idx_cp(wrap(c0 + 3), idx1, si1).start()
        return carry

    lax.fori_loop(0, n_chunks // 2, body, 0)

    e_cp(idx0, rows0, se0).wait()
    b_cp(idx0, bv0, sb0).wait()
    idx_cp(0, idx1, si1).wait()


def kernel(features, labels, emb_table, bias_table, bias):
    B = features.shape[0]
    info = plsc.get_sparse_core_info()
    nc, ns = info.num_cores, info.num_subcores
    nw = nc * ns
    rows_per_w = B // nw
    n_chunks = rows_per_w // C

    feat_flat = features.reshape(-1).astype(jnp.int32)
    bias_flat = bias_table.reshape(-1)

    mesh = plsc.VectorSubcoreMesh(core_axis_name="c", subcore_axis_name="s")
    fm = pl.kernel(
        functools.partial(_fm_kernel, nw=nw, nc=nc,
                          rows_per_w=rows_per_w, n_chunks=n_chunks),
        mesh=mesh,
        compiler_params=pltpu.CompilerParams(use_tc_tiling_on_sc=False),
        out_type=jax.ShapeDtypeStruct((B,), jnp.float32),
        scratch_types=[
            pltpu.VMEM((CF,), jnp.int32),
            pltpu.VMEM((CF,), jnp.int32),
            pltpu.VMEM((CF, K), jnp.float32),
            pltpu.VMEM((CF, K), jnp.float32),
            pltpu.VMEM((CF + 16,), jnp.float32),
            pltpu.VMEM((CF + 16,), jnp.float32),
            pltpu.VMEM((C,), jnp.float32),
            pltpu.SemaphoreType.DMA,
            pltpu.SemaphoreType.DMA,
            pltpu.SemaphoreType.DMA,
            pltpu.SemaphoreType.DMA,
            pltpu.SemaphoreType.DMA,
            pltpu.SemaphoreType.DMA,
        ],
    )
    out = fm(feat_flat, _row_major_table(emb_table), bias_flat)
    return out.reshape(B, 1) + bias
